# baseline (speedup 1.0000x reference)
import functools

import jax
import jax.numpy as jnp
from jax import lax
from jax.experimental import pallas as pl
from jax.experimental.pallas import tpu as pltpu
from jax.experimental.pallas import tpu_sc as plsc

_INFO = plsc.get_sparse_core_info()
_NC = _INFO.num_cores
_NS = _INFO.num_subcores
_NW = _NC * _NS
_IPG = 128
_RING = 4
_LAG = 2


def _make_gather(V, D, B):
  n = B // _NW // _IPG
  assert B % (_NW * _IPG) == 0 and n % _RING == 0 and n >= 2 * _RING
  assert D % 32 == 0
  dw = D // 2
  mesh = plsc.VectorSubcoreMesh(core_axis_name="c", subcore_axis_name="s")

  @functools.partial(
      pl.kernel,
      mesh=mesh,
      compiler_params=pltpu.CompilerParams(use_tc_tiling_on_sc=False),
      out_type=jax.ShapeDtypeStruct((B, D), jnp.int32),
      scratch_types=(
          [pltpu.VMEM((n, _IPG), jnp.int32)]
          + [pltpu.VMEM((_IPG, dw), jnp.int32)] * _RING
          + [pltpu.VMEM((_IPG, D), jnp.int32)] * _RING
          + [pltpu.SemaphoreType.DMA] * (2 * _RING)
      ),
  )
  def k(table_hbm, idx_hbm, out_hbm, idx_all, *scratch):
    gbuf = scratch[:_RING]
    fbuf = scratch[_RING:2 * _RING]
    s_g = scratch[2 * _RING:3 * _RING]
    s_w = scratch[3 * _RING:]
    wid = lax.axis_index("s") * _NC + lax.axis_index("c")
    row0 = wid * n

    pltpu.sync_copy(idx_hbm.at[pl.ds(row0, n)], idx_all)

    def out_chunk(j):
      return out_hbm.at[pl.ds((row0 + j) * _IPG, _IPG)]

    def fire(j, p):
      pltpu.async_copy(table_hbm.at[idx_all.at[j]], gbuf[p], s_g[p])

    def wait_gather(p):
      pltpu.make_async_copy(table_hbm.at[idx_all.at[0]], gbuf[p],
                            s_g[p]).wait()

    def convert(p):
      hi = jnp.int32(-65536)

      def crow(r, carry):
        for g in range(dw // 16):
          x = gbuf[p][r, pl.ds(g * 16, 16)]
          a = lax.shift_left(x, 16)
          b = lax.bitwise_and(x, hi)
          fbuf[p][r, pl.ds(g * 16, 16)] = a
          fbuf[p][r, pl.ds(D // 2 + g * 16, 16)] = b
        return carry

      lax.fori_loop(0, _IPG, crow, 0)

    def wait_wb(p):
      pltpu.make_async_copy(fbuf[p], out_chunk(0), s_w[p]).wait()

    fire(0, 0)
    fire(1, 1)
    for r in range(_RING):
      wait_gather(r)
      fire(r + _LAG, (r + _LAG) % _RING)
      convert(r)
      pltpu.async_copy(fbuf[r], out_chunk(r), s_w[r])

    def body(g, carry):
      for r in range(_RING):
        i = g * _RING + r
        wait_gather(r)
        fire(i + _LAG, (r + _LAG) % _RING)
        wait_wb(r)
        convert(r)
        pltpu.async_copy(fbuf[r], out_chunk(i), s_w[r])
      return carry

    lax.fori_loop(1, n // _RING - 1, body, 0)

    for r in range(_RING):
      i = n - _RING + r
      wait_gather(r)
      if i + _LAG < n:
        fire(i + _LAG, (r + _LAG) % _RING)
      wait_wb(r)
      convert(r)
      pltpu.async_copy(fbuf[r], out_chunk(i), s_w[r])
    for r in range(_RING):
      wait_wb(r)

  return k


def kernel(token_ids, weights):
  B0, B1 = token_ids.shape
  V, D = weights.shape
  B = B0 * B1
  idx = token_ids.reshape(B // _IPG, _IPG).astype(jnp.int32)
  wb16 = weights.astype(jnp.bfloat16)
  packed = lax.bitcast_convert_type(
      wb16.reshape(V, 2, D // 2).swapaxes(1, 2), jnp.int32)
  out = _make_gather(V, D, B)(packed, idx)
  return lax.bitcast_convert_type(out, jnp.float32).reshape(B0, B1, D)


# speedup vs baseline: 1.3409x; 1.3409x over previous
import functools

import jax
import jax.numpy as jnp
from jax import lax
from jax.experimental import pallas as pl
from jax.experimental.pallas import tpu as pltpu
from jax.experimental.pallas import tpu_sc as plsc

_INFO = plsc.get_sparse_core_info()
_NC = _INFO.num_cores
_NS = _INFO.num_subcores
_NW = _NC * _NS
_IPG = 128
_RING = 4
_LAG = 2


def _make_gather(V, D, B):
  n = B // _NW // _IPG
  assert B % (_NW * _IPG) == 0 and n % _RING == 0 and n >= 2 * _RING
  assert D % 32 == 0
  dw = D // 2
  mesh = plsc.VectorSubcoreMesh(core_axis_name="c", subcore_axis_name="s")

  @functools.partial(
      pl.kernel,
      mesh=mesh,
      compiler_params=pltpu.CompilerParams(use_tc_tiling_on_sc=False),
      out_type=jax.ShapeDtypeStruct((B, D), jnp.int32),
      scratch_types=(
          [pltpu.VMEM((n, _IPG), jnp.int32)]
          + [pltpu.VMEM((_IPG, dw), jnp.int32)] * _RING
          + [pltpu.VMEM((_IPG, D), jnp.int32)] * _RING
          + [pltpu.SemaphoreType.DMA] * (2 * _RING)
      ),
  )
  def k(table_hbm, idx_hbm, out_hbm, idx_all, *scratch):
    gbuf = scratch[:_RING]
    fbuf = scratch[_RING:2 * _RING]
    s_g = scratch[2 * _RING:3 * _RING]
    s_w = scratch[3 * _RING:]
    wid = lax.axis_index("s") * _NC + lax.axis_index("c")
    row0 = wid * n

    pltpu.sync_copy(idx_hbm.at[pl.ds(row0, n)], idx_all)

    def out_chunk(j):
      return out_hbm.at[pl.ds((row0 + j) * _IPG, _IPG)]

    def fire(j, p):
      pltpu.async_copy(table_hbm.at[idx_all.at[j]], gbuf[p], s_g[p])

    def wait_gather(p):
      pltpu.make_async_copy(table_hbm.at[idx_all.at[0]], gbuf[p],
                            s_g[p]).wait()

    def convert(p):
      hi = jnp.int32(-65536)

      @plsc.parallel_loop(0, _IPG, 1, unroll=4)
      def crow(r):
        for g in range(dw // 16):
          x = gbuf[p][r, pl.ds(g * 16, 16)]
          a = lax.shift_left(x, 16)
          b = lax.bitwise_and(x, hi)
          fbuf[p][r, pl.ds(g * 16, 16)] = a
          fbuf[p][r, pl.ds(D // 2 + g * 16, 16)] = b

    def wait_wb(p):
      pltpu.make_async_copy(fbuf[p], out_chunk(0), s_w[p]).wait()

    fire(0, 0)
    fire(1, 1)
    for r in range(_RING):
      wait_gather(r)
      fire(r + _LAG, (r + _LAG) % _RING)
      convert(r)
      pltpu.async_copy(fbuf[r], out_chunk(r), s_w[r])

    def body(g, carry):
      for r in range(_RING):
        i = g * _RING + r
        wait_gather(r)
        fire(i + _LAG, (r + _LAG) % _RING)
        wait_wb(r)
        convert(r)
        pltpu.async_copy(fbuf[r], out_chunk(i), s_w[r])
      return carry

    lax.fori_loop(1, n // _RING - 1, body, 0)

    for r in range(_RING):
      i = n - _RING + r
      wait_gather(r)
      if i + _LAG < n:
        fire(i + _LAG, (r + _LAG) % _RING)
      wait_wb(r)
      convert(r)
      pltpu.async_copy(fbuf[r], out_chunk(i), s_w[r])
    for r in range(_RING):
      wait_wb(r)

  return k


def kernel(token_ids, weights):
  B0, B1 = token_ids.shape
  V, D = weights.shape
  B = B0 * B1
  idx = token_ids.reshape(B // _IPG, _IPG).astype(jnp.int32)
  wb16 = weights.astype(jnp.bfloat16)
  packed = lax.bitcast_convert_type(
      wb16.reshape(V, 2, D // 2).swapaxes(1, 2), jnp.int32)
  out = _make_gather(V, D, B)(packed, idx)
  return lax.bitcast_convert_type(out, jnp.float32).reshape(B0, B1, D)
